# Initial kernel scaffold; baseline (speedup 1.0000x reference)
import functools

import jax
import jax.numpy as jnp
from jax import lax
from jax.experimental import pallas as pl
from jax.experimental.pallas import tpu as pltpu
from jax.experimental.pallas import tpu_sc as plsc

_N = 10000
_E = 320000
_D = 128
_H = 128
_R = 8

_NC = 2
_NS = 16
_NW = _NC * _NS
_LANES = 16

_EDGES_PER_WORKER = _E // _NW
_CHUNK = 80
_NCHUNKS = _EDGES_PER_WORKER // _CHUNK

_TC_BLOCK_N = 1000


def _tc_transform_body(x_ref, wrel_ref, wself_ref, b_ref, h_ref, self_ref):
    xb = x_ref[...]
    for r in range(_R):
        h_ref[r] = jnp.dot(xb, wrel_ref[r], preferred_element_type=jnp.float32)
    self_ref[...] = (
        jnp.dot(xb, wself_ref[...], preferred_element_type=jnp.float32)
        + b_ref[...]
    )


def _tc_transform(x, W_rel, W_self, b2d):
    grid = (_N // _TC_BLOCK_N,)
    return pl.pallas_call(
        _tc_transform_body,
        grid=grid,
        in_specs=[
            pl.BlockSpec((_TC_BLOCK_N, _D), lambda i: (i, 0)),
            pl.BlockSpec((_R, _D, _H), lambda i: (0, 0, 0)),
            pl.BlockSpec((_D, _H), lambda i: (0, 0)),
            pl.BlockSpec((1, _H), lambda i: (0, 0)),
        ],
        out_specs=[
            pl.BlockSpec((_R, _TC_BLOCK_N, _H), lambda i: (0, i, 0)),
            pl.BlockSpec((_TC_BLOCK_N, _H), lambda i: (i, 0)),
        ],
        out_shape=[
            jax.ShapeDtypeStruct((_R, _N, _H), jnp.float32),
            jax.ShapeDtypeStruct((_N, _H), jnp.float32),
        ],
    )(x, W_rel, W_self, b2d)


def _sc_body(h_hbm, src_hbm, dst_hbm, type_hbm, norm_hbm, zero_hbm, out_hbm,
             src_v, dst_v, type_v, norm_v, idx_v, rows_v, sem):
    c = lax.axis_index("c")
    s = lax.axis_index("s")
    wid = c * _NS + s
    rows_per_tile = _N // _NS

    def run(agg):
        pltpu.sync_copy(zero_hbm.at[pl.ds(s * rows_per_tile, rows_per_tile)],
                        agg.at[pl.ds(s * rows_per_tile, rows_per_tile)])
        plsc.subcore_barrier()

        @pl.loop(0, _NCHUNKS)
        def chunk_loop(k):
            base = wid * _EDGES_PER_WORKER + k * _CHUNK
            pltpu.sync_copy(src_hbm.at[pl.ds(base, _CHUNK)], src_v)
            pltpu.sync_copy(type_hbm.at[pl.ds(base, _CHUNK)], type_v)
            pltpu.sync_copy(norm_hbm.at[pl.ds(base, _CHUNK)], norm_v)
            pltpu.sync_copy(dst_hbm.at[pl.ds(base, _CHUNK)], dst_v)
            for g in range(_CHUNK // _LANES):
                sl = pl.ds(g * _LANES, _LANES)
                idx_v[sl] = type_v[sl] * _N + src_v[sl]
            pltpu.async_copy(h_hbm.at[idx_v], rows_v, sem).wait()

            @pl.loop(0, _CHUNK)
            def scale_loop(e):
                bnorm = plsc.load_gather(
                    norm_v, [jnp.full((_LANES,), 0, jnp.int32) + e])
                for j in range(_H // _LANES):
                    sl = pl.ds(j * _LANES, _LANES)
                    rows_v[e, sl] = rows_v[e, sl] * bnorm

            pltpu.sync_copy(rows_v, agg.at[dst_v], add=True)

        plsc.subcore_barrier()
        pltpu.sync_copy(agg.at[pl.ds(s * rows_per_tile, rows_per_tile)],
                        out_hbm.at[c, pl.ds(s * rows_per_tile, rows_per_tile)])

    pl.run_scoped(run, pltpu.VMEM_SHARED((_N, _H), jnp.float32))


def _sc_gather_scatter(h_flat, src, dst, etype, norm, zeros_nh):
    mesh = plsc.VectorSubcoreMesh(core_axis_name="c", subcore_axis_name="s")
    kfn = pl.kernel(
        _sc_body,
        out_type=jax.ShapeDtypeStruct((_NC, _N, _H), jnp.float32),
        mesh=mesh,
        scratch_types=[
            pltpu.VMEM((_CHUNK,), jnp.int32),
            pltpu.VMEM((_CHUNK,), jnp.int32),
            pltpu.VMEM((_CHUNK,), jnp.int32),
            pltpu.VMEM((_CHUNK,), jnp.float32),
            pltpu.VMEM((_CHUNK,), jnp.int32),
            pltpu.VMEM((_CHUNK, _H), jnp.float32),
            pltpu.SemaphoreType.DMA,
        ],
    )
    return kfn(h_flat, src, dst, etype, norm, zeros_nh)


def _tc_combine_body(p_ref, self_ref, o_ref):
    o_ref[...] = jnp.maximum(p_ref[0] + p_ref[1] + self_ref[...], 0.0)


def _tc_combine(partials, self_term):
    grid = (_N // _TC_BLOCK_N,)
    return pl.pallas_call(
        _tc_combine_body,
        grid=grid,
        in_specs=[
            pl.BlockSpec((_NC, _TC_BLOCK_N, _H), lambda i: (0, i, 0)),
            pl.BlockSpec((_TC_BLOCK_N, _H), lambda i: (i, 0)),
        ],
        out_specs=pl.BlockSpec((_TC_BLOCK_N, _H), lambda i: (i, 0)),
        out_shape=jax.ShapeDtypeStruct((_N, _H), jnp.float32),
    )(partials, self_term)


def kernel(x, edge_index, edge_norm, edge_type, W_rel, W_self, b):
    assert x.shape == (_N, _D) and edge_norm.shape == (_E,)
    assert W_rel.shape == (_R, _D, _H)

    h_all, self_term = _tc_transform(x, W_rel, W_self, b.reshape(1, _H))
    h_flat = h_all.reshape(_R * _N, _H)

    src = edge_index[0]
    dst = edge_index[1]
    zeros_nh = jnp.zeros((_N, _H), jnp.float32)

    partials = _sc_gather_scatter(h_flat, src, dst, edge_type, edge_norm,
                                  zeros_nh)
    return _tc_combine(partials, self_term)


# speedup vs baseline: 10.3868x; 10.3868x over previous
import dataclasses
import functools

import jax
import jax.numpy as jnp
from jax import lax
from jax.experimental import pallas as pl
from jax.experimental.pallas import tpu as pltpu
from jax.experimental.pallas import tpu_sc as plsc

_N = 10000
_E = 320000
_D = 128
_H = 128
_R = 8

_NC = 2
_NS = 16
_NW = _NC * _NS
_LANES = 16

_EDGES_PER_WORKER = _E // _NW
_CHUNK = 80
_NCHUNKS = _EDGES_PER_WORKER // _CHUNK

_ROWS_PER_TILE = 640
_N_PAD = _ROWS_PER_TILE * _NS

_TC_BLOCK_N = 1000


def _tc_transform_body(x_ref, wrel_ref, wself_ref, b_ref, h_ref, self_ref):
    xb = x_ref[...]
    for r in range(_R):
        h_ref[r] = jnp.dot(xb, wrel_ref[r], preferred_element_type=jnp.float32)
    self_ref[...] = (
        jnp.dot(xb, wself_ref[...], preferred_element_type=jnp.float32)
        + b_ref[...]
    )


def _tc_transform(x, W_rel, W_self, b2d):
    grid = (_N // _TC_BLOCK_N,)
    return pl.pallas_call(
        _tc_transform_body,
        grid=grid,
        in_specs=[
            pl.BlockSpec((_TC_BLOCK_N, _D), lambda i: (i, 0)),
            pl.BlockSpec((_R, _D, _H), lambda i: (0, 0, 0)),
            pl.BlockSpec((_D, _H), lambda i: (0, 0)),
            pl.BlockSpec((1, _H), lambda i: (0, 0)),
        ],
        out_specs=[
            pl.BlockSpec((_R, _TC_BLOCK_N, _H), lambda i: (0, i, 0)),
            pl.BlockSpec((_TC_BLOCK_N, _H), lambda i: (i, 0)),
        ],
        out_shape=[
            jax.ShapeDtypeStruct((_R, _N, _H), jnp.float32),
            jax.ShapeDtypeStruct((_N, _H), jnp.float32),
        ],
    )(x, W_rel, W_self, b2d)


def _sc_body(h_hbm, src_hbm, dst_hbm, type_hbm, norm_hbm, zero_hbm, out_hbm,
             src_v, dst_v, type_v, norm_v, idx_v, rows_v, agg, sem):
    c = lax.axis_index("c")
    s = lax.axis_index("s")
    wid = c * _NS + s

    if True:
        pltpu.sync_copy(zero_hbm,
                        agg.at[pl.ds(s * _ROWS_PER_TILE, _ROWS_PER_TILE)])
        plsc.subcore_barrier()

        @pl.loop(0, _NCHUNKS)
        def chunk_loop(k):
            base = wid * _EDGES_PER_WORKER + k * _CHUNK
            pltpu.sync_copy(src_hbm.at[pl.ds(base, _CHUNK)], src_v)
            pltpu.sync_copy(type_hbm.at[pl.ds(base, _CHUNK)], type_v)
            pltpu.sync_copy(norm_hbm.at[pl.ds(base, _CHUNK)], norm_v)
            pltpu.sync_copy(dst_hbm.at[pl.ds(base, _CHUNK)], dst_v)
            for g in range(_CHUNK // _LANES):
                sl = pl.ds(g * _LANES, _LANES)
                idx_v[sl] = type_v[sl] * _N + src_v[sl]
            pltpu.async_copy(h_hbm.at[idx_v], rows_v, sem).wait()

            @pl.loop(0, _CHUNK)
            def scale_loop(e):
                bnorm = plsc.load_gather(
                    norm_v, [jnp.full((_LANES,), 0, jnp.int32) + e])
                for j in range(_H // _LANES):
                    sl = pl.ds(j * _LANES, _LANES)
                    rows_v[e, sl] = rows_v[e, sl] * bnorm

            pltpu.sync_copy(rows_v, agg.at[dst_v], add=True)

        plsc.subcore_barrier()
        pltpu.sync_copy(agg.at[pl.ds(s * _ROWS_PER_TILE, _ROWS_PER_TILE)],
                        out_hbm.at[c, pl.ds(s * _ROWS_PER_TILE, _ROWS_PER_TILE)])


def _sc_gather_scatter(h_flat, src, dst, etype, norm, zeros_nh):
    mesh = plsc.VectorSubcoreMesh(core_axis_name="c", subcore_axis_name="s")
    cp = pltpu.CompilerParams()
    if "needs_layout_passes" in pltpu.CompilerParams.__dataclass_fields__:
        cp = dataclasses.replace(cp, needs_layout_passes=False)
    kfn = pl.kernel(
        _sc_body,
        out_type=jax.ShapeDtypeStruct((_NC, _N_PAD, _H), jnp.float32),
        mesh=mesh,
        scratch_types=[
            pltpu.VMEM((_CHUNK,), jnp.int32),
            pltpu.VMEM((_CHUNK,), jnp.int32),
            pltpu.VMEM((_CHUNK,), jnp.int32),
            pltpu.VMEM((_CHUNK,), jnp.float32),
            pltpu.VMEM((_CHUNK,), jnp.int32),
            pltpu.VMEM((_CHUNK, _H), jnp.float32),
            pltpu.VMEM_SHARED((_N_PAD, _H), jnp.float32),
            pltpu.SemaphoreType.DMA,
        ],
        compiler_params=cp,
    )
    return kfn(h_flat, src, dst, etype, norm, zeros_nh)


def _tc_combine_body(p_ref, self_ref, o_ref):
    o_ref[...] = jnp.maximum(p_ref[0] + p_ref[1] + self_ref[...], 0.0)


def _tc_combine(partials, self_term):
    grid = (_N // _TC_BLOCK_N,)
    return pl.pallas_call(
        _tc_combine_body,
        grid=grid,
        in_specs=[
            pl.BlockSpec((_NC, _TC_BLOCK_N, _H), lambda i: (0, i, 0)),
            pl.BlockSpec((_TC_BLOCK_N, _H), lambda i: (i, 0)),
        ],
        out_specs=pl.BlockSpec((_TC_BLOCK_N, _H), lambda i: (i, 0)),
        out_shape=jax.ShapeDtypeStruct((_N, _H), jnp.float32),
    )(partials, self_term)


def kernel(x, edge_index, edge_norm, edge_type, W_rel, W_self, b):
    assert x.shape == (_N, _D) and edge_norm.shape == (_E,)
    assert W_rel.shape == (_R, _D, _H)

    h_all, self_term = _tc_transform(x, W_rel, W_self, b.reshape(1, _H))
    h_flat = h_all.reshape(_R * _N, _H)

    src = edge_index[0]
    dst = edge_index[1]
    zeros_nh = jnp.zeros((_ROWS_PER_TILE, _H), jnp.float32)

    partials = _sc_gather_scatter(h_flat, src, dst, edge_type, edge_norm,
                                  zeros_nh)
    return _tc_combine(partials, self_term)


# speedup vs baseline: 12.4508x; 1.1987x over previous
import dataclasses
import functools

import jax
import jax.numpy as jnp
from jax import lax
from jax.experimental import pallas as pl
from jax.experimental.pallas import tpu as pltpu
from jax.experimental.pallas import tpu_sc as plsc

_N = 10000
_E = 320000
_D = 128
_H = 128
_R = 8

_NC = 2
_NS = 16
_NW = _NC * _NS
_LANES = 16

_EDGES_PER_WORKER = _E // _NW
_CHUNK = 80
_NCHUNKS = _EDGES_PER_WORKER // _CHUNK

_ROWS_PER_TILE = 640
_N_PAD = _ROWS_PER_TILE * _NS

_TC_BLOCK_N = 1000


def _tc_transform_body(x_ref, wrel_ref, wself_ref, b_ref, h_ref, self_ref):
    xb = x_ref[...]
    for r in range(_R):
        h_ref[r] = jnp.dot(xb, wrel_ref[r], preferred_element_type=jnp.float32)
    self_ref[...] = (
        jnp.dot(xb, wself_ref[...], preferred_element_type=jnp.float32)
        + b_ref[...]
    )


def _tc_transform(x, W_rel, W_self, b2d):
    grid = (_N // _TC_BLOCK_N,)
    return pl.pallas_call(
        _tc_transform_body,
        grid=grid,
        in_specs=[
            pl.BlockSpec((_TC_BLOCK_N, _D), lambda i: (i, 0)),
            pl.BlockSpec((_R, _D, _H), lambda i: (0, 0, 0)),
            pl.BlockSpec((_D, _H), lambda i: (0, 0)),
            pl.BlockSpec((1, _H), lambda i: (0, 0)),
        ],
        out_specs=[
            pl.BlockSpec((_R, _TC_BLOCK_N, _H), lambda i: (0, i, 0)),
            pl.BlockSpec((_TC_BLOCK_N, _H), lambda i: (i, 0)),
        ],
        out_shape=[
            jax.ShapeDtypeStruct((_R, _N, _H), jnp.float32),
            jax.ShapeDtypeStruct((_N, _H), jnp.float32),
        ],
    )(x, W_rel, W_self, b2d)


def _sc_body(h_hbm, src_hbm, dst_hbm, type_hbm, norm_hbm, zero_hbm, out_hbm,
             src_a, type_a,
             idx_c0, idx_c1, dst_c0, dst_c1, norm_c0, norm_c1, rows0, rows1,
             agg, gsem0, gsem1, ssem0, ssem1, msem0, msem1, dsem0, dsem1):
    c = lax.axis_index("c")
    s = lax.axis_index("s")
    wid = c * _NS + s
    shard = wid * _EDGES_PER_WORKER

    pltpu.sync_copy(zero_hbm,
                    agg.at[pl.ds(s * _ROWS_PER_TILE, _ROWS_PER_TILE)])
    plsc.subcore_barrier()

    def prep(k, idx_c, dst_c, norm_c, msem, dsem):
        pltpu.async_copy(dst_hbm.at[pl.ds(shard + k * _CHUNK, _CHUNK)],
                         dst_c, dsem)
        pltpu.async_copy(norm_hbm.at[pl.ds(shard + k * _CHUNK, _CHUNK)],
                         norm_c, msem)
        pltpu.sync_copy(src_hbm.at[pl.ds(shard + k * _CHUNK, _CHUNK)], src_a)
        pltpu.sync_copy(type_hbm.at[pl.ds(shard + k * _CHUNK, _CHUNK)], type_a)
        for g in range(_CHUNK // _LANES):
            sl = pl.ds(g * _LANES, _LANES)
            idx_c[sl] = type_a[sl] * _N + src_a[sl]

    def gather_start(idx_c, rows, gsem):
        pltpu.async_copy(h_hbm.at[idx_c], rows, gsem)

    def gather_wait(idx_c, rows, gsem):
        pltpu.make_async_copy(h_hbm.at[idx_c], rows, gsem).wait()

    def meta_wait(k, dst_c, norm_c, msem, dsem):
        pltpu.make_async_copy(
            dst_hbm.at[pl.ds(shard + k * _CHUNK, _CHUNK)], dst_c, dsem).wait()
        pltpu.make_async_copy(
            norm_hbm.at[pl.ds(shard + k * _CHUNK, _CHUNK)], norm_c,
            msem).wait()

    def scale(norm_c, rows):
        ez = lax.shift_right_logical(src_a[pl.ds(0, _LANES)], 31)
        for e in range(_CHUNK):
            bnorm = plsc.load_gather(
                norm_c, [ez + jnp.full((_LANES,), e, jnp.int32)])
            for j in range(_H // _LANES):
                sl = pl.ds(j * _LANES, _LANES)
                rows[e, sl] = rows[e, sl] * bnorm

    def scatter(rows, dst_c):
        pltpu.sync_copy(rows, agg.at[dst_c], add=True)

    prep(0, idx_c0, dst_c0, norm_c0, msem0, dsem0)
    gather_start(idx_c0, rows0, gsem0)

    gather_wait(idx_c0, rows0, gsem0)
    meta_wait(0, dst_c0, norm_c0, msem0, dsem0)
    prep(1, idx_c1, dst_c1, norm_c1, msem1, dsem1)
    gather_start(idx_c1, rows1, gsem1)
    scale(norm_c0, rows0)
    scatter(rows0, dst_c0)

    @pl.loop(0, (_NCHUNKS - 1) // 2)
    def pair_loop(t):
        k1 = 1 + 2 * t
        gather_wait(idx_c1, rows1, gsem1)
        meta_wait(k1, dst_c1, norm_c1, msem1, dsem1)
        prep(k1 + 1, idx_c0, dst_c0, norm_c0, msem0, dsem0)
        gather_start(idx_c0, rows0, gsem0)
        scale(norm_c1, rows1)
        scatter(rows1, dst_c1)
        gather_wait(idx_c0, rows0, gsem0)
        meta_wait(k1 + 1, dst_c0, norm_c0, msem0, dsem0)
        @pl.when(t < (_NCHUNKS - 1) // 2 - 1)
        def _():
            prep(k1 + 2, idx_c1, dst_c1, norm_c1, msem1, dsem1)
            gather_start(idx_c1, rows1, gsem1)
        scale(norm_c0, rows0)
        scatter(rows0, dst_c0)

    plsc.subcore_barrier()
    pltpu.sync_copy(agg.at[pl.ds(s * _ROWS_PER_TILE, _ROWS_PER_TILE)],
                    out_hbm.at[c, pl.ds(s * _ROWS_PER_TILE, _ROWS_PER_TILE)])


def _sc_gather_scatter(h_flat, src, dst, etype, norm, zeros_nh):
    mesh = plsc.VectorSubcoreMesh(core_axis_name="c", subcore_axis_name="s")
    cp = pltpu.CompilerParams()
    if "needs_layout_passes" in pltpu.CompilerParams.__dataclass_fields__:
        cp = dataclasses.replace(cp, needs_layout_passes=False)
    kfn = pl.kernel(
        _sc_body,
        out_type=jax.ShapeDtypeStruct((_NC, _N_PAD, _H), jnp.float32),
        mesh=mesh,
        scratch_types=[
            pltpu.VMEM((_CHUNK,), jnp.int32),
            pltpu.VMEM((_CHUNK,), jnp.int32),
            pltpu.VMEM((_CHUNK,), jnp.int32),
            pltpu.VMEM((_CHUNK,), jnp.int32),
            pltpu.VMEM((_CHUNK,), jnp.int32),
            pltpu.VMEM((_CHUNK,), jnp.int32),
            pltpu.VMEM((_CHUNK,), jnp.float32),
            pltpu.VMEM((_CHUNK,), jnp.float32),
            pltpu.VMEM((_CHUNK, _H), jnp.float32),
            pltpu.VMEM((_CHUNK, _H), jnp.float32),
            pltpu.VMEM_SHARED((_N_PAD, _H), jnp.float32),
            pltpu.SemaphoreType.DMA,
            pltpu.SemaphoreType.DMA,
            pltpu.SemaphoreType.DMA,
            pltpu.SemaphoreType.DMA,
            pltpu.SemaphoreType.DMA,
            pltpu.SemaphoreType.DMA,
            pltpu.SemaphoreType.DMA,
            pltpu.SemaphoreType.DMA,
        ],
        compiler_params=cp,
    )
    return kfn(h_flat, src, dst, etype, norm, zeros_nh)


def _tc_combine_body(p_ref, self_ref, o_ref):
    o_ref[...] = jnp.maximum(p_ref[0] + p_ref[1] + self_ref[...], 0.0)


def _tc_combine(partials, self_term):
    grid = (_N // _TC_BLOCK_N,)
    return pl.pallas_call(
        _tc_combine_body,
        grid=grid,
        in_specs=[
            pl.BlockSpec((_NC, _TC_BLOCK_N, _H), lambda i: (0, i, 0)),
            pl.BlockSpec((_TC_BLOCK_N, _H), lambda i: (i, 0)),
        ],
        out_specs=pl.BlockSpec((_TC_BLOCK_N, _H), lambda i: (i, 0)),
        out_shape=jax.ShapeDtypeStruct((_N, _H), jnp.float32),
    )(partials, self_term)


def kernel(x, edge_index, edge_norm, edge_type, W_rel, W_self, b):
    assert x.shape == (_N, _D) and edge_norm.shape == (_E,)
    assert W_rel.shape == (_R, _D, _H)

    h_all, self_term = _tc_transform(x, W_rel, W_self, b.reshape(1, _H))
    h_flat = h_all.reshape(_R * _N, _H)

    src = edge_index[0]
    dst = edge_index[1]
    zeros_nh = jnp.zeros((_ROWS_PER_TILE, _H), jnp.float32)

    partials = _sc_gather_scatter(h_flat, src, dst, edge_type, edge_norm,
                                  zeros_nh)
    return _tc_combine(partials, self_term)


# speedup vs baseline: 16.7899x; 1.3485x over previous
import dataclasses
import functools

import jax
import jax.numpy as jnp
from jax import lax
from jax.experimental import pallas as pl
from jax.experimental.pallas import tpu as pltpu
from jax.experimental.pallas import tpu_sc as plsc

_N = 10000
_E = 320000
_D = 128
_H = 128
_R = 8

_NC = 2
_NS = 16
_NW = _NC * _NS
_LANES = 16

_EDGES_PER_WORKER = _E // _NW
_CHUNK = 80
_NCHUNKS = _EDGES_PER_WORKER // _CHUNK

_ROWS_PER_TILE = 640
_N_PAD = _ROWS_PER_TILE * _NS

_TC_BLOCK_N = 1000


def _tc_transform_body(x_ref, wrel_ref, wself_ref, b_ref, h_ref, self_ref):
    xb = x_ref[...]
    for r in range(_R):
        h_ref[r] = jnp.dot(xb, wrel_ref[r], preferred_element_type=jnp.float32)
    self_ref[...] = (
        jnp.dot(xb, wself_ref[...], preferred_element_type=jnp.float32)
        + b_ref[...]
    )


def _tc_transform(x, W_rel, W_self, b2d):
    grid = (_N // _TC_BLOCK_N,)
    return pl.pallas_call(
        _tc_transform_body,
        grid=grid,
        in_specs=[
            pl.BlockSpec((_TC_BLOCK_N, _D), lambda i: (i, 0)),
            pl.BlockSpec((_R, _D, _H), lambda i: (0, 0, 0)),
            pl.BlockSpec((_D, _H), lambda i: (0, 0)),
            pl.BlockSpec((1, _H), lambda i: (0, 0)),
        ],
        out_specs=[
            pl.BlockSpec((_R, _TC_BLOCK_N, _H), lambda i: (0, i, 0)),
            pl.BlockSpec((_TC_BLOCK_N, _H), lambda i: (i, 0)),
        ],
        out_shape=[
            jax.ShapeDtypeStruct((_R, _N, _H), jnp.float32),
            jax.ShapeDtypeStruct((_N, _H), jnp.float32),
        ],
    )(x, W_rel, W_self, b2d)


def _gidx_body(src_ref, type_ref, o_ref):
    o_ref[...] = type_ref[...] * _N + src_ref[...]


def _tc_gidx(src2d, type2d):
    return pl.pallas_call(
        _gidx_body,
        out_shape=jax.ShapeDtypeStruct((_E // 128, 128), jnp.int32),
    )(src2d, type2d)


def _sc_body(h_hbm, gidx_hbm, dst_hbm, norm_hbm, zero_hbm, out_hbm,
             gidx_a, norm_a, idx_c0, idx_c1, dst_c0, dst_c1,
             rows0, rows1,
             agg, gsem0, gsem1, dsem0, dsem1):
    c = lax.axis_index("c")
    s = lax.axis_index("s")
    wid = c * _NS + s
    shard = wid * _EDGES_PER_WORKER

    pltpu.sync_copy(zero_hbm,
                    agg.at[pl.ds(s * _ROWS_PER_TILE, _ROWS_PER_TILE)])
    pltpu.sync_copy(gidx_hbm.at[pl.ds(shard, _EDGES_PER_WORKER)], gidx_a)
    pltpu.sync_copy(norm_hbm.at[pl.ds(shard, _EDGES_PER_WORKER)], norm_a)
    plsc.subcore_barrier()

    def prep(k, idx_c, dst_c, dsem):
        pltpu.async_copy(dst_hbm.at[pl.ds(shard + k * _CHUNK, _CHUNK)],
                         dst_c, dsem)
        for g in range(_CHUNK // _LANES):
            sl = pl.ds(g * _LANES, _LANES)
            sa = pl.ds(k * _CHUNK + g * _LANES, _LANES)
            idx_c[sl] = gidx_a[sa]

    def gather_start(idx_c, rows, gsem):
        pltpu.async_copy(h_hbm.at[idx_c], rows, gsem)

    def gather_wait(idx_c, rows, gsem):
        pltpu.make_async_copy(h_hbm.at[idx_c], rows, gsem).wait()

    def meta_wait(k, dst_c, dsem):
        pltpu.make_async_copy(
            dst_hbm.at[pl.ds(shard + k * _CHUNK, _CHUNK)], dst_c, dsem).wait()

    def scale(k, rows):
        ez = lax.shift_right_logical(gidx_a[pl.ds(0, _LANES)], 31)
        base = k * _CHUNK
        for e in range(_CHUNK):
            bnorm = plsc.load_gather(
                norm_a, [ez + (base + jnp.full((_LANES,), e, jnp.int32))])
            for j in range(_H // _LANES):
                sl = pl.ds(j * _LANES, _LANES)
                rows[e, sl] = rows[e, sl] * bnorm

    def scatter(rows, dst_c):
        pltpu.sync_copy(rows, agg.at[dst_c], add=True)

    prep(0, idx_c0, dst_c0, dsem0)
    gather_start(idx_c0, rows0, gsem0)

    gather_wait(idx_c0, rows0, gsem0)
    meta_wait(0, dst_c0, dsem0)
    prep(1, idx_c1, dst_c1, dsem1)
    gather_start(idx_c1, rows1, gsem1)
    scale(0, rows0)
    scatter(rows0, dst_c0)

    @pl.loop(0, (_NCHUNKS - 1) // 2)
    def pair_loop(t):
        k1 = 1 + 2 * t
        gather_wait(idx_c1, rows1, gsem1)
        meta_wait(k1, dst_c1, dsem1)
        prep(k1 + 1, idx_c0, dst_c0, dsem0)
        gather_start(idx_c0, rows0, gsem0)
        scale(k1, rows1)
        scatter(rows1, dst_c1)
        gather_wait(idx_c0, rows0, gsem0)
        meta_wait(k1 + 1, dst_c0, dsem0)
        @pl.when(t < (_NCHUNKS - 1) // 2 - 1)
        def _():
            prep(k1 + 2, idx_c1, dst_c1, dsem1)
            gather_start(idx_c1, rows1, gsem1)
        scale(k1 + 1, rows0)
        scatter(rows0, dst_c0)

    plsc.subcore_barrier()
    pltpu.sync_copy(agg.at[pl.ds(s * _ROWS_PER_TILE, _ROWS_PER_TILE)],
                    out_hbm.at[c, pl.ds(s * _ROWS_PER_TILE, _ROWS_PER_TILE)])


def _sc_gather_scatter(h_flat, gidx, dst, norm, zeros_nh):
    mesh = plsc.VectorSubcoreMesh(core_axis_name="c", subcore_axis_name="s")
    cp = pltpu.CompilerParams()
    if "needs_layout_passes" in pltpu.CompilerParams.__dataclass_fields__:
        cp = dataclasses.replace(cp, needs_layout_passes=False)
    kfn = pl.kernel(
        _sc_body,
        out_type=jax.ShapeDtypeStruct((_NC, _N_PAD, _H), jnp.float32),
        mesh=mesh,
        scratch_types=[
            pltpu.VMEM((_EDGES_PER_WORKER,), jnp.int32),
            pltpu.VMEM((_EDGES_PER_WORKER,), jnp.float32),
            pltpu.VMEM((_CHUNK,), jnp.int32),
            pltpu.VMEM((_CHUNK,), jnp.int32),
            pltpu.VMEM((_CHUNK,), jnp.int32),
            pltpu.VMEM((_CHUNK,), jnp.int32),
            pltpu.VMEM((_CHUNK, _H), jnp.float32),
            pltpu.VMEM((_CHUNK, _H), jnp.float32),
            pltpu.VMEM_SHARED((_N_PAD, _H), jnp.float32),
            pltpu.SemaphoreType.DMA,
            pltpu.SemaphoreType.DMA,
            pltpu.SemaphoreType.DMA,
            pltpu.SemaphoreType.DMA,
        ],
        compiler_params=cp,
    )
    return kfn(h_flat, gidx, dst, norm, zeros_nh)


def _tc_combine_body(p_ref, self_ref, o_ref):
    o_ref[...] = jnp.maximum(p_ref[0] + p_ref[1] + self_ref[...], 0.0)


def _tc_combine(partials, self_term):
    grid = (_N // _TC_BLOCK_N,)
    return pl.pallas_call(
        _tc_combine_body,
        grid=grid,
        in_specs=[
            pl.BlockSpec((_NC, _TC_BLOCK_N, _H), lambda i: (0, i, 0)),
            pl.BlockSpec((_TC_BLOCK_N, _H), lambda i: (i, 0)),
        ],
        out_specs=pl.BlockSpec((_TC_BLOCK_N, _H), lambda i: (i, 0)),
        out_shape=jax.ShapeDtypeStruct((_N, _H), jnp.float32),
    )(partials, self_term)


def kernel(x, edge_index, edge_norm, edge_type, W_rel, W_self, b):
    assert x.shape == (_N, _D) and edge_norm.shape == (_E,)
    assert W_rel.shape == (_R, _D, _H)

    h_all, self_term = _tc_transform(x, W_rel, W_self, b.reshape(1, _H))
    h_flat = h_all.reshape(_R * _N, _H)

    src = edge_index[0]
    dst = edge_index[1]
    gidx = _tc_gidx(src.reshape(_E // 128, 128),
                    edge_type.reshape(_E // 128, 128)).reshape(_E)
    zeros_nh = jnp.zeros((_ROWS_PER_TILE, _H), jnp.float32)

    partials = _sc_gather_scatter(h_flat, gidx, dst, edge_norm, zeros_nh)
    return _tc_combine(partials, self_term)
